# baseline (speedup 1.0000x reference)
import functools

import jax
import jax.numpy as jnp
from jax import lax
from jax.experimental import pallas as pl
from jax.experimental.pallas import tpu as pltpu
from jax.experimental.pallas import tpu_sc as plsc

NC = 2
NS = 16
NW = NC * NS
L = 16

B = 4
NV = 4096
FG = 8192
FP = 8192
EU = 3 * FP


def _edge_prep(pred_faces):
    f = pred_faces.astype(jnp.int32)
    u = jnp.concatenate([f[:, 0], f[:, 1], f[:, 2]], axis=0)
    v = jnp.concatenate([f[:, 1], f[:, 2], f[:, 0]], axis=0)
    key = jnp.minimum(u, v) * NV + jnp.maximum(u, v)
    skey = jnp.sort(key)
    first = jnp.concatenate(
        [jnp.ones((1,), dtype=bool), skey[1:] != skey[:-1]])
    ea = skey >> 12
    eb = skey & (NV - 1)
    return ea, eb, first.astype(jnp.float32), jnp.sum(first)


def _normals_sc(gv3, gf_flat):
    fpt = FG // NW
    mesh = plsc.VectorSubcoreMesh(core_axis_name="c", subcore_axis_name="s")

    @functools.partial(
        pl.kernel,
        out_type=jax.ShapeDtypeStruct((NC * 3 * NV,), jnp.float32),
        mesh=mesh,
        compiler_params=pltpu.CompilerParams(needs_layout_passes=False),
        scratch_types=dict(
            f0=pltpu.VMEM((fpt,), jnp.int32),
            f1=pltpu.VMEM((fpt,), jnp.int32),
            f2=pltpu.VMEM((fpt,), jnp.int32),
            fnx=pltpu.VMEM((fpt,), jnp.float32),
            fny=pltpu.VMEM((fpt,), jnp.float32),
            fnz=pltpu.VMEM((fpt,), jnp.float32),
            vtx=pltpu.VMEM((NV,), jnp.float32),
            vty=pltpu.VMEM((NV,), jnp.float32),
            vtz=pltpu.VMEM((NV,), jnp.float32),
            zbuf=pltpu.VMEM((NV // NS,), jnp.float32),
            shx=pltpu.VMEM_SHARED((NV,), jnp.float32),
            shy=pltpu.VMEM_SHARED((NV,), jnp.float32),
            shz=pltpu.VMEM_SHARED((NV,), jnp.float32),
            sem=pltpu.SemaphoreType.DMA,
            sem2=pltpu.SemaphoreType.DMA,
        ),
    )
    def k(gv_hbm, gf_hbm, out_hbm, f0, f1, f2, fnx, fny, fnz,
          vtx, vty, vtz, zbuf, shx, shy, shz, sem, sem2):
        core = lax.axis_index("c")
        sid = lax.axis_index("s")
        wid = sid * NC + core
        nsl = NV // NS

        tdescs = [pltpu.async_copy(gv_hbm.at[pl.ds(c * NV, NV)], t, sem)
                  for c, t in enumerate((vtx, vty, vtz))]
        fdescs = [pltpu.async_copy(gf_hbm.at[pl.ds(k_ * FG + wid * fpt, fpt)],
                                   f, sem2)
                  for k_, f in enumerate((f0, f1, f2))]

        for i in range(nsl // L):
            zbuf[pl.ds(i * L, L)] = jnp.zeros((L,), jnp.float32)
        pltpu.sync_copy(zbuf, shx.at[pl.ds(sid * nsl, nsl)])
        pltpu.sync_copy(zbuf, shy.at[pl.ds(sid * nsl, nsl)])
        pltpu.sync_copy(zbuf, shz.at[pl.ds(sid * nsl, nsl)])
        for d in tdescs + fdescs:
            d.wait()
        plsc.subcore_barrier()

        def face_chunk(j, _):
            sl = pl.ds(j * L, L)
            a = f0[sl]
            bb = f1[sl]
            cc = f2[sl]
            v0x = plsc.load_gather(vtx, [a])
            v0y = plsc.load_gather(vty, [a])
            v0z = plsc.load_gather(vtz, [a])
            v1x = plsc.load_gather(vtx, [bb])
            v1y = plsc.load_gather(vty, [bb])
            v1z = plsc.load_gather(vtz, [bb])
            v2x = plsc.load_gather(vtx, [cc])
            v2y = plsc.load_gather(vty, [cc])
            v2z = plsc.load_gather(vtz, [cc])
            e1x, e1y, e1z = v1x - v0x, v1y - v0y, v1z - v0z
            e2x, e2y, e2z = v2x - v0x, v2y - v0y, v2z - v0z
            fnx[sl] = e1y * e2z - e1z * e2y
            fny[sl] = e1z * e2x - e1x * e2z
            fnz[sl] = e1x * e2y - e1y * e2x
            return 0

        lax.fori_loop(0, fpt // L, face_chunk, 0)

        adds = []
        for fidx in (f0, f1, f2):
            adds.append(pltpu.async_copy(fnx, shx.at[fidx], sem2, add=True))
            adds.append(pltpu.async_copy(fny, shy.at[fidx], sem2, add=True))
            adds.append(pltpu.async_copy(fnz, shz.at[fidx], sem2, add=True))
        for d in adds:
            d.wait()
        plsc.subcore_barrier()

        sl = pl.ds(sid * nsl, nsl)
        base = core * 3 * NV + sid * nsl
        pltpu.sync_copy(shx.at[sl], out_hbm.at[pl.ds(base + 0 * NV, nsl)])
        pltpu.sync_copy(shy.at[sl], out_hbm.at[pl.ds(base + 1 * NV, nsl)])
        pltpu.sync_copy(shz.at[sl], out_hbm.at[pl.ds(base + 2 * NV, nsl)])

    return k(gv3, gf_flat)


BP = 1024
NVC = 1024


def _chamfer_tc(pv, gvt, nrm):
    def body(pv_ref, gvt_ref, nrm_ref, idx_ref, nhat_ref):
        p = pv_ref[...]
        g = gvt_ref[...]
        g2 = jnp.sum(g * g, axis=0, keepdims=True)
        p2 = jnp.sum(p * p, axis=1, keepdims=True)
        bf = jnp.bfloat16
        f32 = jnp.float32
        rg = -2.0 * g
        rgh = rg.astype(bf)
        rgl = (rg - rgh.astype(f32)).astype(bf)
        g2h = g2.astype(bf)
        g2l = (g2 - g2h.astype(f32)).astype(bf)
        ph = p.astype(bf)
        pl_ = (p - ph.astype(f32)).astype(bf)
        p2h = p2.astype(bf)
        p2l = (p2 - p2h.astype(f32)).astype(bf)
        ones_c = jnp.ones((BP, 1), bf)
        zeros_c = jnp.zeros((BP, 3), bf)
        lhs16 = jnp.concatenate(
            [ph, ph, pl_, ones_c, ones_c, p2h, p2l, zeros_c], axis=1)
        ones_r = jnp.ones((1, NV), bf)
        zeros_r = jnp.zeros((3, NV), bf)
        rhs16 = jnp.concatenate(
            [rgh, rgl, rgh, g2h, g2l, ones_r, ones_r, zeros_r], axis=0)
        dn = (((1,), (0,)), ((), ()))
        ids = lax.broadcasted_iota(jnp.int32, (BP, NVC), 1)
        mks = []
        for h in range(NV // NVC):
            d = lax.dot_general(lhs16, rhs16[:, h * NVC:(h + 1) * NVC], dn,
                                preferred_element_type=jnp.float32)
            bits = lax.bitcast_convert_type(d, jnp.int32)
            key = lax.bitcast_convert_type((bits & ~(NVC - 1)) | ids,
                                           jnp.float32)
            mks.append(jnp.min(key, axis=1, keepdims=True))
        m = mks[0]
        for mk in mks[1:]:
            m = jnp.minimum(m, mk)
        amin = None
        for h, mk in enumerate(mks):
            lid = lax.bitcast_convert_type(mk, jnp.int32) & (NVC - 1)
            cand = jnp.where(mk <= m, lid + h * NVC, NV)
            amin = cand if amin is None else jnp.minimum(amin, cand)
        idx_ref[0] = amin

        n = nrm_ref[0] + nrm_ref[1]
        ns = jnp.sqrt(jnp.sum(n * n, axis=0, keepdims=True))
        nhat_ref[...] = n / jnp.maximum(ns, 1e-12)

    return pl.pallas_call(
        body,
        grid=(NV // BP,),
        in_specs=[
            pl.BlockSpec((BP, 3), lambda i: (i, 0)),
            pl.BlockSpec((3, NV), lambda i: (0, 0)),
            pl.BlockSpec((2, 3, BP), lambda i: (0, 0, i)),
        ],
        out_specs=[
            pl.BlockSpec((1, BP, 1), lambda i: (i, 0, 0)),
            pl.BlockSpec((3, BP), lambda i: (0, i)),
        ],
        out_shape=[
            jax.ShapeDtypeStruct((NV // BP, BP, 1), jnp.int32),
            jax.ShapeDtypeStruct((3, NV), jnp.float32),
        ],
    )(pv, gvt, nrm)


def _edge_loss_sc(ea, eb, ew, idx, pvt3, nhat3):
    epw = EU // NW
    mesh = plsc.VectorSubcoreMesh(core_axis_name="c", subcore_axis_name="s")

    @functools.partial(
        pl.kernel,
        out_type=jax.ShapeDtypeStruct((NW * L,), jnp.float32),
        mesh=mesh,
        compiler_params=pltpu.CompilerParams(needs_layout_passes=False),
        scratch_types=dict(
            eas=pltpu.VMEM((epw,), jnp.int32),
            ebs=pltpu.VMEM((epw,), jnp.int32),
            ews=pltpu.VMEM((epw,), jnp.float32),
            idxt=pltpu.VMEM((NV,), jnp.int32),
            px=pltpu.VMEM((NV,), jnp.float32),
            py=pltpu.VMEM((NV,), jnp.float32),
            pz=pltpu.VMEM((NV,), jnp.float32),
            nx=pltpu.VMEM((NV,), jnp.float32),
            ny=pltpu.VMEM((NV,), jnp.float32),
            nz=pltpu.VMEM((NV,), jnp.float32),
            accb=pltpu.VMEM((L,), jnp.float32),
            sem=pltpu.SemaphoreType.DMA,
        ),
    )
    def k(ea_hbm, eb_hbm, ew_hbm, idx_hbm, pv_hbm, nh_hbm, out_hbm,
          eas, ebs, ews, idxt, px, py, pz, nx, ny, nz, accb, sem):
        core = lax.axis_index("c")
        sid = lax.axis_index("s")
        wid = sid * NC + core
        base = wid * epw
        descs = [
            pltpu.async_copy(ea_hbm.at[pl.ds(base, epw)], eas, sem),
            pltpu.async_copy(eb_hbm.at[pl.ds(base, epw)], ebs, sem),
            pltpu.async_copy(ew_hbm.at[pl.ds(base, epw)], ews, sem),
            pltpu.async_copy(idx_hbm, idxt, sem),
            pltpu.async_copy(pv_hbm.at[pl.ds(0 * NV, NV)], px, sem),
            pltpu.async_copy(pv_hbm.at[pl.ds(1 * NV, NV)], py, sem),
            pltpu.async_copy(pv_hbm.at[pl.ds(2 * NV, NV)], pz, sem),
            pltpu.async_copy(nh_hbm.at[pl.ds(0 * NV, NV)], nx, sem),
            pltpu.async_copy(nh_hbm.at[pl.ds(1 * NV, NV)], ny, sem),
            pltpu.async_copy(nh_hbm.at[pl.ds(2 * NV, NV)], nz, sem),
        ]
        for d in descs:
            d.wait()

        def edge_chunk(j, acc):
            sl = pl.ds(j * L, L)
            a = eas[sl]
            bb = ebs[sl]
            w = ews[sl]
            ia = plsc.load_gather(idxt, [a])
            ib = plsc.load_gather(idxt, [bb])
            dx = plsc.load_gather(px, [a]) - plsc.load_gather(px, [bb])
            dy = plsc.load_gather(py, [a]) - plsc.load_gather(py, [bb])
            dz = plsc.load_gather(pz, [a]) - plsc.load_gather(pz, [bb])
            d1 = dx * plsc.load_gather(nx, [ia]) \
                + dy * plsc.load_gather(ny, [ia]) \
                + dz * plsc.load_gather(nz, [ia])
            d2 = dx * plsc.load_gather(nx, [ib]) \
                + dy * plsc.load_gather(ny, [ib]) \
                + dz * plsc.load_gather(nz, [ib])
            return acc + (jnp.abs(d1) + jnp.abs(d2)) * w

        acc = lax.fori_loop(0, epw // L, edge_chunk,
                            jnp.zeros((L,), jnp.float32))
        accb[...] = acc
        pltpu.sync_copy(accb, out_hbm.at[pl.ds(wid * L, L)])

    return k(ea, eb, ew, idx, pvt3, nhat3)


def kernel(pred_vertices, pred_faces, gt_vertices, gt_faces):
    pred_vertices = pred_vertices.astype(jnp.float32)
    gt_vertices = gt_vertices.astype(jnp.float32)

    ea, eb, ew, nuniq = _edge_prep(pred_faces)

    gvt = jnp.transpose(gt_vertices, (0, 2, 1))
    pvt = jnp.transpose(pred_vertices, (0, 2, 1))
    gf_flat = jnp.transpose(gt_faces.astype(jnp.int32), (1, 0)).reshape(-1)

    total = jnp.zeros((), jnp.float32)
    for b in range(B):
        nrm_b = _normals_sc(gvt[b].reshape(-1), gf_flat)
        idx_b, nhat_b = _chamfer_tc(pred_vertices[b], gvt[b],
                                    nrm_b.reshape(2, 3, NV))
        parts_b = _edge_loss_sc(ea, eb, ew, idx_b.reshape(-1),
                                pvt[b].reshape(-1), nhat_b.reshape(-1))
        total = total + jnp.sum(parts_b)

    denom = (B * 2 * nuniq).astype(jnp.float32)
    return total / denom


# speedup vs baseline: 1.1926x; 1.1926x over previous
import functools

import jax
import jax.numpy as jnp
from jax import lax
from jax.experimental import pallas as pl
from jax.experimental.pallas import tpu as pltpu
from jax.experimental.pallas import tpu_sc as plsc

NC = 2
NS = 16
NW = NC * NS
L = 16

B = 4
NV = 4096
FG = 8192
FP = 8192
EU = 3 * FP


def _edge_prep(pred_faces):
    f = pred_faces.astype(jnp.int32)
    u = jnp.concatenate([f[:, 0], f[:, 1], f[:, 2]], axis=0)
    v = jnp.concatenate([f[:, 1], f[:, 2], f[:, 0]], axis=0)
    key = jnp.minimum(u, v) * NV + jnp.maximum(u, v)
    skey = key
    first = jnp.concatenate(
        [jnp.ones((1,), dtype=bool), skey[1:] != skey[:-1]])
    ea = skey >> 12
    eb = skey & (NV - 1)
    return ea, eb, first.astype(jnp.float32), jnp.sum(first)


def _normals_sc(gv_flat, gf_flat):
    fpt = FG // NS
    mesh = plsc.VectorSubcoreMesh(core_axis_name="c", subcore_axis_name="s")

    @functools.partial(
        pl.kernel,
        out_type=jax.ShapeDtypeStruct((B * 3 * NV,), jnp.float32),
        mesh=mesh,
        compiler_params=pltpu.CompilerParams(needs_layout_passes=False),
        scratch_types=dict(
            f0=pltpu.VMEM((fpt,), jnp.int32),
            f1=pltpu.VMEM((fpt,), jnp.int32),
            f2=pltpu.VMEM((fpt,), jnp.int32),
            fnx=pltpu.VMEM((fpt,), jnp.float32),
            fny=pltpu.VMEM((fpt,), jnp.float32),
            fnz=pltpu.VMEM((fpt,), jnp.float32),
            vtx0=pltpu.VMEM((NV,), jnp.float32),
            vty0=pltpu.VMEM((NV,), jnp.float32),
            vtz0=pltpu.VMEM((NV,), jnp.float32),
            vtx1=pltpu.VMEM((NV,), jnp.float32),
            vty1=pltpu.VMEM((NV,), jnp.float32),
            vtz1=pltpu.VMEM((NV,), jnp.float32),
            zbuf=pltpu.VMEM((NV // NS,), jnp.float32),
            shx=pltpu.VMEM_SHARED((NV,), jnp.float32),
            shy=pltpu.VMEM_SHARED((NV,), jnp.float32),
            shz=pltpu.VMEM_SHARED((NV,), jnp.float32),
            sem=pltpu.SemaphoreType.DMA,
            sem2=pltpu.SemaphoreType.DMA,
        ),
    )
    def k(gv_hbm, gf_hbm, out_hbm, f0, f1, f2, fnx, fny, fnz,
          vtx0, vty0, vtz0, vtx1, vty1, vtz1, zbuf, shx, shy, shz,
          sem, sem2):
        core = lax.axis_index("c")
        sid = lax.axis_index("s")
        nsl = NV // NS
        vt = [(vtx0, vty0, vtz0), (vtx1, vty1, vtz1)]

        pltpu.sync_copy(gf_hbm.at[pl.ds(0 * FG + sid * fpt, fpt)], f0)
        pltpu.sync_copy(gf_hbm.at[pl.ds(1 * FG + sid * fpt, fpt)], f1)
        pltpu.sync_copy(gf_hbm.at[pl.ds(2 * FG + sid * fpt, fpt)], f2)

        for i in range(nsl // L):
            zbuf[pl.ds(i * L, L)] = jnp.zeros((L,), jnp.float32)

        def issue(lb):
            row = (core * 2 + lb) * 3
            return [pltpu.async_copy(gv_hbm.at[pl.ds((row + c) * NV, NV)],
                                     vt[lb][c], sem) for c in range(3)]

        descs = issue(0)
        for lb in range(2):
            b = core * 2 + lb
            row = b * 3
            pltpu.sync_copy(zbuf, shx.at[pl.ds(sid * nsl, nsl)])
            pltpu.sync_copy(zbuf, shy.at[pl.ds(sid * nsl, nsl)])
            pltpu.sync_copy(zbuf, shz.at[pl.ds(sid * nsl, nsl)])
            for d in descs:
                d.wait()
            if lb == 0:
                descs = issue(1)
            vtx, vty, vtz = vt[lb]
            plsc.subcore_barrier()

            def face_chunk(j, _):
                sl = pl.ds(j * L, L)
                a = f0[sl]
                bb = f1[sl]
                cc = f2[sl]
                v0x = plsc.load_gather(vtx, [a])
                v0y = plsc.load_gather(vty, [a])
                v0z = plsc.load_gather(vtz, [a])
                v1x = plsc.load_gather(vtx, [bb])
                v1y = plsc.load_gather(vty, [bb])
                v1z = plsc.load_gather(vtz, [bb])
                v2x = plsc.load_gather(vtx, [cc])
                v2y = plsc.load_gather(vty, [cc])
                v2z = plsc.load_gather(vtz, [cc])
                e1x, e1y, e1z = v1x - v0x, v1y - v0y, v1z - v0z
                e2x, e2y, e2z = v2x - v0x, v2y - v0y, v2z - v0z
                fnx[sl] = e1y * e2z - e1z * e2y
                fny[sl] = e1z * e2x - e1x * e2z
                fnz[sl] = e1x * e2y - e1y * e2x
                return 0

            lax.fori_loop(0, fpt // L, face_chunk, 0)

            adds = []
            for fidx in (f0, f1, f2):
                adds.append(pltpu.async_copy(fnx, shx.at[fidx], sem2,
                                             add=True))
                adds.append(pltpu.async_copy(fny, shy.at[fidx], sem2,
                                             add=True))
                adds.append(pltpu.async_copy(fnz, shz.at[fidx], sem2,
                                             add=True))
            for d in adds:
                d.wait()
            plsc.subcore_barrier()

            sl = pl.ds(sid * nsl, nsl)
            pltpu.sync_copy(shx.at[sl],
                            out_hbm.at[pl.ds((row + 0) * NV + sid * nsl, nsl)])
            pltpu.sync_copy(shy.at[sl],
                            out_hbm.at[pl.ds((row + 1) * NV + sid * nsl, nsl)])
            pltpu.sync_copy(shz.at[sl],
                            out_hbm.at[pl.ds((row + 2) * NV + sid * nsl, nsl)])
            plsc.subcore_barrier()

    return k(gv_flat, gf_flat)


BP = 1024
NVC = 1024


def _chamfer_tc(pv, gvt, nrm):
    def body(pv_ref, gvt_ref, nrm_ref, idx_ref, nhat_ref):
        p = pv_ref[0]
        g = gvt_ref[0]
        g2 = jnp.sum(g * g, axis=0, keepdims=True)
        p2 = jnp.sum(p * p, axis=1, keepdims=True)
        bf = jnp.bfloat16
        f32 = jnp.float32
        rg = -2.0 * g
        rgh = rg.astype(bf)
        rgl = (rg - rgh.astype(f32)).astype(bf)
        g2h = g2.astype(bf)
        g2l = (g2 - g2h.astype(f32)).astype(bf)
        ph = p.astype(bf)
        pl_ = (p - ph.astype(f32)).astype(bf)
        p2h = p2.astype(bf)
        p2l = (p2 - p2h.astype(f32)).astype(bf)
        ones_c = jnp.ones((BP, 1), bf)
        zeros_c = jnp.zeros((BP, 3), bf)
        lhs16 = jnp.concatenate(
            [ph, ph, pl_, ones_c, ones_c, p2h, p2l, zeros_c], axis=1)
        ones_r = jnp.ones((1, NV), bf)
        zeros_r = jnp.zeros((3, NV), bf)
        rhs16 = jnp.concatenate(
            [rgh, rgl, rgh, g2h, g2l, ones_r, ones_r, zeros_r], axis=0)
        dn = (((1,), (0,)), ((), ()))
        ids = lax.broadcasted_iota(jnp.int32, (BP, NVC), 1)
        mks = []
        for h in range(NV // NVC):
            d = lax.dot_general(lhs16, rhs16[:, h * NVC:(h + 1) * NVC], dn,
                                preferred_element_type=jnp.float32)
            bits = lax.bitcast_convert_type(d, jnp.int32)
            key = lax.bitcast_convert_type((bits & ~(NVC - 1)) | ids, jnp.float32)
            mks.append(jnp.min(key, axis=1, keepdims=True))
        m = mks[0]
        for mk in mks[1:]:
            m = jnp.minimum(m, mk)
        amin = None
        for h, mk in enumerate(mks):
            lid = lax.bitcast_convert_type(mk, jnp.int32) & (NVC - 1)
            cand = jnp.where(mk <= m, lid + h * NVC, NV)
            amin = cand if amin is None else jnp.minimum(amin, cand)
        idx_ref[0, 0] = amin

        n = nrm_ref[0]
        ns = jnp.sqrt(jnp.sum(n * n, axis=0, keepdims=True))
        nhat_ref[0] = n / jnp.maximum(ns, 1e-12)

    return pl.pallas_call(
        body,
        grid=(B, NV // BP),
        in_specs=[
            pl.BlockSpec((1, BP, 3), lambda b, i: (b, i, 0)),
            pl.BlockSpec((1, 3, NV), lambda b, i: (b, 0, 0)),
            pl.BlockSpec((1, 3, BP), lambda b, i: (b, 0, i)),
        ],
        out_specs=[
            pl.BlockSpec((1, 1, BP, 1), lambda b, i: (b, i, 0, 0)),
            pl.BlockSpec((1, 3, BP), lambda b, i: (b, 0, i)),
        ],
        out_shape=[
            jax.ShapeDtypeStruct((B, NV // BP, BP, 1), jnp.int32),
            jax.ShapeDtypeStruct((B, 3, NV), jnp.float32),
        ],
    )(pv, gvt, nrm)


def _edge_loss_sc(ea, eb, ew, idx, pvt_flat, nhat_flat):
    epw = EU // NW
    mesh = plsc.VectorSubcoreMesh(core_axis_name="c", subcore_axis_name="s")

    tabs = [('idxt', jnp.int32), ('px', jnp.float32), ('py', jnp.float32),
            ('pz', jnp.float32), ('nx', jnp.float32), ('ny', jnp.float32),
            ('nz', jnp.float32)]
    scratch = dict(
        eas=pltpu.VMEM((epw,), jnp.int32),
        ebs=pltpu.VMEM((epw,), jnp.int32),
        ews=pltpu.VMEM((epw,), jnp.float32),
        accb=pltpu.VMEM((L,), jnp.float32),
        sem=pltpu.SemaphoreType.DMA,
    )
    for nm, dt in tabs:
        scratch[nm + '0'] = pltpu.VMEM((NV,), dt)
        scratch[nm + '1'] = pltpu.VMEM((NV,), dt)

    @functools.partial(
        pl.kernel,
        out_type=jax.ShapeDtypeStruct((NW * L,), jnp.float32),
        mesh=mesh,
        compiler_params=pltpu.CompilerParams(needs_layout_passes=False),
        scratch_types=scratch,
    )
    def k(ea_hbm, eb_hbm, ew_hbm, idx_hbm, pv_hbm, nh_hbm, out_hbm,
          eas, ebs, ews, accb, sem, **bufs):
        core = lax.axis_index("c")
        sid = lax.axis_index("s")
        wid = sid * NC + core
        base = wid * epw
        pltpu.sync_copy(ea_hbm.at[pl.ds(base, epw)], eas)
        pltpu.sync_copy(eb_hbm.at[pl.ds(base, epw)], ebs)
        pltpu.sync_copy(ew_hbm.at[pl.ds(base, epw)], ews)

        def issue(b):
            pb = b % 2
            row = b * 3
            srcs = [idx_hbm.at[pl.ds(b * NV, NV)]] + [
                hbm.at[pl.ds((row + c) * NV, NV)]
                for hbm, cs in ((pv_hbm, (0, 1, 2)), (nh_hbm, (0, 1, 2)))
                for c in cs]
            descs = []
            for (nm, _), s in zip(tabs, srcs):
                descs.append(pltpu.async_copy(s, bufs[f'{nm}{pb}'], sem))
            return descs

        descs = issue(0)
        acc = jnp.zeros((L,), jnp.float32)
        for b in range(B):
            for d in descs:
                d.wait()
            if b + 1 < B:
                descs = issue(b + 1)
            pb = b % 2
            idxt = bufs[f'idxt{pb}']
            px, py, pz = bufs[f'px{pb}'], bufs[f'py{pb}'], bufs[f'pz{pb}']
            nx, ny, nz = bufs[f'nx{pb}'], bufs[f'ny{pb}'], bufs[f'nz{pb}']

            def edge_chunk(j, acc):
                sl = pl.ds(j * L, L)
                a = eas[sl]
                bb = ebs[sl]
                w = ews[sl]
                ia = plsc.load_gather(idxt, [a])
                ib = plsc.load_gather(idxt, [bb])
                dx = plsc.load_gather(px, [a]) - plsc.load_gather(px, [bb])
                dy = plsc.load_gather(py, [a]) - plsc.load_gather(py, [bb])
                dz = plsc.load_gather(pz, [a]) - plsc.load_gather(pz, [bb])
                d1 = dx * plsc.load_gather(nx, [ia]) \
                    + dy * plsc.load_gather(ny, [ia]) \
                    + dz * plsc.load_gather(nz, [ia])
                d2 = dx * plsc.load_gather(nx, [ib]) \
                    + dy * plsc.load_gather(ny, [ib]) \
                    + dz * plsc.load_gather(nz, [ib])
                return acc + (jnp.abs(d1) + jnp.abs(d2)) * w

            acc = lax.fori_loop(0, epw // L, edge_chunk, acc)

        accb[...] = acc
        pltpu.sync_copy(accb, out_hbm.at[pl.ds(wid * L, L)])

    return k(ea, eb, ew, idx, pvt_flat, nhat_flat)


def kernel(pred_vertices, pred_faces, gt_vertices, gt_faces):
    pred_vertices = pred_vertices.astype(jnp.float32)
    gt_vertices = gt_vertices.astype(jnp.float32)

    ea, eb, ew, nuniq = _edge_prep(pred_faces)

    gv_flat = jnp.transpose(gt_vertices, (0, 2, 1)).reshape(-1)
    pvt = jnp.transpose(pred_vertices, (0, 2, 1))
    pvt_flat = pvt.reshape(-1)
    gf_flat = jnp.transpose(gt_faces.astype(jnp.int32), (1, 0)).reshape(-1)

    nrm_flat = _normals_sc(gv_flat, gf_flat)
    gvt = jnp.transpose(gt_vertices, (0, 2, 1))
    idx3, nhat = _chamfer_tc(pred_vertices, gvt, nrm_flat.reshape(B, 3, NV))
    partials = _edge_loss_sc(ea, eb, ew, idx3.reshape(-1),
                             pvt_flat, nhat.reshape(-1))

    denom = (B * 2 * nuniq).astype(jnp.float32)
    return jnp.sum(partials) / denom


# speedup vs baseline: 1.2384x; 1.0384x over previous
import functools

import jax
import jax.numpy as jnp
from jax import lax
from jax.experimental import pallas as pl
from jax.experimental.pallas import tpu as pltpu
from jax.experimental.pallas import tpu_sc as plsc

NC = 2
NS = 16
NW = NC * NS
L = 16

B = 4
NV = 4096
FG = 8192
FP = 8192
EU = 3 * FP


def _edge_prep(pred_faces):
    f = pred_faces.astype(jnp.int32)
    u = jnp.concatenate([f[:, 0], f[:, 1], f[:, 2]], axis=0)
    v = jnp.concatenate([f[:, 1], f[:, 2], f[:, 0]], axis=0)
    key = jnp.minimum(u, v) * NV + jnp.maximum(u, v)
    skey = jnp.sort(key)
    first = jnp.concatenate(
        [jnp.ones((1,), dtype=bool), skey[1:] != skey[:-1]])
    ea = skey >> 12
    eb = skey & (NV - 1)
    return ea, eb, first.astype(jnp.float32), jnp.sum(first)


def _normals_sc(gv_flat, gf_flat):
    fpt = FG // NS
    mesh = plsc.VectorSubcoreMesh(core_axis_name="c", subcore_axis_name="s")

    @functools.partial(
        pl.kernel,
        out_type=jax.ShapeDtypeStruct((B * 3 * NV,), jnp.float32),
        mesh=mesh,
        compiler_params=pltpu.CompilerParams(needs_layout_passes=False),
        scratch_types=dict(
            f0=pltpu.VMEM((fpt,), jnp.int32),
            f1=pltpu.VMEM((fpt,), jnp.int32),
            f2=pltpu.VMEM((fpt,), jnp.int32),
            fnx=pltpu.VMEM((fpt,), jnp.float32),
            fny=pltpu.VMEM((fpt,), jnp.float32),
            fnz=pltpu.VMEM((fpt,), jnp.float32),
            vtx0=pltpu.VMEM((NV,), jnp.float32),
            vty0=pltpu.VMEM((NV,), jnp.float32),
            vtz0=pltpu.VMEM((NV,), jnp.float32),
            vtx1=pltpu.VMEM((NV,), jnp.float32),
            vty1=pltpu.VMEM((NV,), jnp.float32),
            vtz1=pltpu.VMEM((NV,), jnp.float32),
            zbuf=pltpu.VMEM((NV // NS,), jnp.float32),
            shx=pltpu.VMEM_SHARED((NV,), jnp.float32),
            shy=pltpu.VMEM_SHARED((NV,), jnp.float32),
            shz=pltpu.VMEM_SHARED((NV,), jnp.float32),
            sem=pltpu.SemaphoreType.DMA,
            sem2=pltpu.SemaphoreType.DMA,
        ),
    )
    def k(gv_hbm, gf_hbm, out_hbm, f0, f1, f2, fnx, fny, fnz,
          vtx0, vty0, vtz0, vtx1, vty1, vtz1, zbuf, shx, shy, shz,
          sem, sem2):
        core = lax.axis_index("c")
        sid = lax.axis_index("s")
        nsl = NV // NS
        vt = [(vtx0, vty0, vtz0), (vtx1, vty1, vtz1)]

        pltpu.sync_copy(gf_hbm.at[pl.ds(0 * FG + sid * fpt, fpt)], f0)
        pltpu.sync_copy(gf_hbm.at[pl.ds(1 * FG + sid * fpt, fpt)], f1)
        pltpu.sync_copy(gf_hbm.at[pl.ds(2 * FG + sid * fpt, fpt)], f2)

        for i in range(nsl // L):
            zbuf[pl.ds(i * L, L)] = jnp.zeros((L,), jnp.float32)

        def issue(lb):
            row = (core * 2 + lb) * 3
            return [pltpu.async_copy(gv_hbm.at[pl.ds((row + c) * NV, NV)],
                                     vt[lb][c], sem) for c in range(3)]

        descs = issue(0)
        for lb in range(2):
            b = core * 2 + lb
            row = b * 3
            pltpu.sync_copy(zbuf, shx.at[pl.ds(sid * nsl, nsl)])
            pltpu.sync_copy(zbuf, shy.at[pl.ds(sid * nsl, nsl)])
            pltpu.sync_copy(zbuf, shz.at[pl.ds(sid * nsl, nsl)])
            for d in descs:
                d.wait()
            if lb == 0:
                descs = issue(1)
            vtx, vty, vtz = vt[lb]
            plsc.subcore_barrier()

            def face_chunk(j, _):
                sl = pl.ds(j * L, L)
                a = f0[sl]
                bb = f1[sl]
                cc = f2[sl]
                v0x = plsc.load_gather(vtx, [a])
                v0y = plsc.load_gather(vty, [a])
                v0z = plsc.load_gather(vtz, [a])
                v1x = plsc.load_gather(vtx, [bb])
                v1y = plsc.load_gather(vty, [bb])
                v1z = plsc.load_gather(vtz, [bb])
                v2x = plsc.load_gather(vtx, [cc])
                v2y = plsc.load_gather(vty, [cc])
                v2z = plsc.load_gather(vtz, [cc])
                e1x, e1y, e1z = v1x - v0x, v1y - v0y, v1z - v0z
                e2x, e2y, e2z = v2x - v0x, v2y - v0y, v2z - v0z
                fnx[sl] = e1y * e2z - e1z * e2y
                fny[sl] = e1z * e2x - e1x * e2z
                fnz[sl] = e1x * e2y - e1y * e2x
                return 0

            lax.fori_loop(0, fpt // L, face_chunk, 0)

            adds = []
            for fidx in (f0, f1, f2):
                adds.append(pltpu.async_copy(fnx, shx.at[fidx], sem2,
                                             add=True))
                adds.append(pltpu.async_copy(fny, shy.at[fidx], sem2,
                                             add=True))
                adds.append(pltpu.async_copy(fnz, shz.at[fidx], sem2,
                                             add=True))
            for d in adds:
                d.wait()
            plsc.subcore_barrier()

            sl = pl.ds(sid * nsl, nsl)
            pltpu.sync_copy(shx.at[sl],
                            out_hbm.at[pl.ds((row + 0) * NV + sid * nsl, nsl)])
            pltpu.sync_copy(shy.at[sl],
                            out_hbm.at[pl.ds((row + 1) * NV + sid * nsl, nsl)])
            pltpu.sync_copy(shz.at[sl],
                            out_hbm.at[pl.ds((row + 2) * NV + sid * nsl, nsl)])
            plsc.subcore_barrier()

    return k(gv_flat, gf_flat)


BP = 1024
NVC = 2048


def _chamfer_tc(pv, gvt, nrm):
    def body(pv_ref, gvt_ref, nrm_ref, idx_ref, nhat_ref):
        p = pv_ref[0]
        g = gvt_ref[0]
        g2 = jnp.sum(g * g, axis=0, keepdims=True)
        p2 = jnp.sum(p * p, axis=1, keepdims=True)
        bf = jnp.bfloat16
        f32 = jnp.float32
        rg = -2.0 * g
        rgh = rg.astype(bf)
        rgl = (rg - rgh.astype(f32)).astype(bf)
        g2h = g2.astype(bf)
        g2l = (g2 - g2h.astype(f32)).astype(bf)
        ph = p.astype(bf)
        pl_ = (p - ph.astype(f32)).astype(bf)
        p2h = p2.astype(bf)
        p2l = (p2 - p2h.astype(f32)).astype(bf)
        ones_c = jnp.ones((BP, 1), bf)
        zeros_c = jnp.zeros((BP, 3), bf)
        lhs16 = jnp.concatenate(
            [ph, ph, pl_, ones_c, ones_c, p2h, p2l, zeros_c], axis=1)
        ones_r = jnp.ones((1, NV), bf)
        zeros_r = jnp.zeros((3, NV), bf)
        rhs16 = jnp.concatenate(
            [rgh, rgl, rgh, g2h, g2l, ones_r, ones_r, zeros_r], axis=0)
        dn = (((1,), (0,)), ((), ()))
        ids = lax.broadcasted_iota(jnp.int32, (BP, NVC), 1)
        mks = []
        for h in range(NV // NVC):
            d = lax.dot_general(lhs16, rhs16[:, h * NVC:(h + 1) * NVC], dn,
                                preferred_element_type=jnp.float32)
            bits = lax.bitcast_convert_type(d, jnp.int32)
            key = lax.bitcast_convert_type((bits & ~(NVC - 1)) | ids, jnp.float32)
            mks.append(jnp.min(key, axis=1, keepdims=True))
        m = mks[0]
        for mk in mks[1:]:
            m = jnp.minimum(m, mk)
        amin = None
        for h, mk in enumerate(mks):
            lid = lax.bitcast_convert_type(mk, jnp.int32) & (NVC - 1)
            cand = jnp.where(mk <= m, lid + h * NVC, NV)
            amin = cand if amin is None else jnp.minimum(amin, cand)
        idx_ref[0, 0] = jnp.transpose(amin, (1, 0))

        n = nrm_ref[0]
        ns = jnp.sqrt(jnp.sum(n * n, axis=0, keepdims=True))
        nhat_ref[0] = n / jnp.maximum(ns, 1e-12)

    return pl.pallas_call(
        body,
        grid=(B, NV // BP),
        in_specs=[
            pl.BlockSpec((1, BP, 3), lambda b, i: (b, i, 0)),
            pl.BlockSpec((1, 3, NV), lambda b, i: (b, 0, 0)),
            pl.BlockSpec((1, 3, BP), lambda b, i: (b, 0, i)),
        ],
        out_specs=[
            pl.BlockSpec((1, 1, 1, BP), lambda b, i: (b, i, 0, 0)),
            pl.BlockSpec((1, 3, BP), lambda b, i: (b, 0, i)),
        ],
        out_shape=[
            jax.ShapeDtypeStruct((B, NV // BP, 1, BP), jnp.int32),
            jax.ShapeDtypeStruct((B, 3, NV), jnp.float32),
        ],
    )(pv, gvt, nrm)


def _edge_loss_sc(ea, eb, ew, idx, pvt_flat, nhat_flat):
    epw = EU // NW
    mesh = plsc.VectorSubcoreMesh(core_axis_name="c", subcore_axis_name="s")

    tabs = [('idxt', jnp.int32), ('px', jnp.float32), ('py', jnp.float32),
            ('pz', jnp.float32), ('nx', jnp.float32), ('ny', jnp.float32),
            ('nz', jnp.float32)]
    scratch = dict(
        eas=pltpu.VMEM((epw,), jnp.int32),
        ebs=pltpu.VMEM((epw,), jnp.int32),
        ews=pltpu.VMEM((epw,), jnp.float32),
        accb=pltpu.VMEM((L,), jnp.float32),
        sem=pltpu.SemaphoreType.DMA,
    )
    for nm, dt in tabs:
        scratch[nm + '0'] = pltpu.VMEM((NV,), dt)
        scratch[nm + '1'] = pltpu.VMEM((NV,), dt)

    @functools.partial(
        pl.kernel,
        out_type=jax.ShapeDtypeStruct((NW * L,), jnp.float32),
        mesh=mesh,
        compiler_params=pltpu.CompilerParams(needs_layout_passes=False),
        scratch_types=scratch,
    )
    def k(ea_hbm, eb_hbm, ew_hbm, idx_hbm, pv_hbm, nh_hbm, out_hbm,
          eas, ebs, ews, accb, sem, **bufs):
        core = lax.axis_index("c")
        sid = lax.axis_index("s")
        wid = sid * NC + core
        base = wid * epw
        pltpu.sync_copy(ea_hbm.at[pl.ds(base, epw)], eas)
        pltpu.sync_copy(eb_hbm.at[pl.ds(base, epw)], ebs)
        pltpu.sync_copy(ew_hbm.at[pl.ds(base, epw)], ews)

        def issue(b):
            pb = b % 2
            row = b * 3
            srcs = [idx_hbm.at[pl.ds(b * NV, NV)]] + [
                hbm.at[pl.ds((row + c) * NV, NV)]
                for hbm, cs in ((pv_hbm, (0, 1, 2)), (nh_hbm, (0, 1, 2)))
                for c in cs]
            descs = []
            for (nm, _), s in zip(tabs, srcs):
                descs.append(pltpu.async_copy(s, bufs[f'{nm}{pb}'], sem))
            return descs

        descs = issue(0)
        acc = jnp.zeros((L,), jnp.float32)
        for b in range(B):
            for d in descs:
                d.wait()
            if b + 1 < B:
                descs = issue(b + 1)
            pb = b % 2
            idxt = bufs[f'idxt{pb}']
            px, py, pz = bufs[f'px{pb}'], bufs[f'py{pb}'], bufs[f'pz{pb}']
            nx, ny, nz = bufs[f'nx{pb}'], bufs[f'ny{pb}'], bufs[f'nz{pb}']

            def edge_chunk(j, acc):
                sl = pl.ds(j * L, L)
                a = eas[sl]
                bb = ebs[sl]
                w = ews[sl]
                ia = plsc.load_gather(idxt, [a])
                ib = plsc.load_gather(idxt, [bb])
                dx = plsc.load_gather(px, [a]) - plsc.load_gather(px, [bb])
                dy = plsc.load_gather(py, [a]) - plsc.load_gather(py, [bb])
                dz = plsc.load_gather(pz, [a]) - plsc.load_gather(pz, [bb])
                d1 = dx * plsc.load_gather(nx, [ia]) \
                    + dy * plsc.load_gather(ny, [ia]) \
                    + dz * plsc.load_gather(nz, [ia])
                d2 = dx * plsc.load_gather(nx, [ib]) \
                    + dy * plsc.load_gather(ny, [ib]) \
                    + dz * plsc.load_gather(nz, [ib])
                return acc + (jnp.abs(d1) + jnp.abs(d2)) * w

            acc = lax.fori_loop(0, epw // L, edge_chunk, acc)

        accb[...] = acc
        pltpu.sync_copy(accb, out_hbm.at[pl.ds(wid * L, L)])

    return k(ea, eb, ew, idx, pvt_flat, nhat_flat)


def kernel(pred_vertices, pred_faces, gt_vertices, gt_faces):
    pred_vertices = pred_vertices.astype(jnp.float32)
    gt_vertices = gt_vertices.astype(jnp.float32)

    ea, eb, ew, nuniq = _edge_prep(pred_faces)

    gv_flat = jnp.transpose(gt_vertices, (0, 2, 1)).reshape(-1)
    pvt = jnp.transpose(pred_vertices, (0, 2, 1))
    pvt_flat = pvt.reshape(-1)
    gf_flat = jnp.transpose(gt_faces.astype(jnp.int32), (1, 0)).reshape(-1)

    nrm_flat = _normals_sc(gv_flat, gf_flat)
    gvt = jnp.transpose(gt_vertices, (0, 2, 1))
    idx3, nhat = _chamfer_tc(pred_vertices, gvt, nrm_flat.reshape(B, 3, NV))
    partials = _edge_loss_sc(ea, eb, ew, idx3.reshape(-1),
                             pvt_flat, nhat.reshape(-1))

    denom = (B * 2 * nuniq).astype(jnp.float32)
    return jnp.sum(partials) / denom


# speedup vs baseline: 1.2910x; 1.0425x over previous
import functools

import jax
import jax.numpy as jnp
from jax import lax
from jax.experimental import pallas as pl
from jax.experimental.pallas import tpu as pltpu
from jax.experimental.pallas import tpu_sc as plsc

NC = 2
NS = 16
NW = NC * NS
L = 16

B = 4
NV = 4096
FG = 8192
FP = 8192
EU = 3 * FP


def _edge_prep(pred_faces):
    f = pred_faces.astype(jnp.int32)
    u = jnp.concatenate([f[:, 0], f[:, 1], f[:, 2]], axis=0)
    v = jnp.concatenate([f[:, 1], f[:, 2], f[:, 0]], axis=0)
    key = jnp.minimum(u, v) * NV + jnp.maximum(u, v)
    skey = jnp.sort(key)
    first = jnp.concatenate(
        [jnp.ones((1,), dtype=bool), skey[1:] != skey[:-1]])
    ea = skey >> 12
    eb = skey & (NV - 1)
    return ea, eb, first.astype(jnp.float32), jnp.sum(first)


def _normals_sc(gv_flat, gf_flat):
    fpt = FG // NS
    mesh = plsc.VectorSubcoreMesh(core_axis_name="c", subcore_axis_name="s")

    @functools.partial(
        pl.kernel,
        out_type=jax.ShapeDtypeStruct((B * 3 * NV,), jnp.float32),
        mesh=mesh,
        compiler_params=pltpu.CompilerParams(needs_layout_passes=False),
        scratch_types=dict(
            f0=pltpu.VMEM((fpt,), jnp.int32),
            f1=pltpu.VMEM((fpt,), jnp.int32),
            f2=pltpu.VMEM((fpt,), jnp.int32),
            fnx=pltpu.VMEM((fpt,), jnp.float32),
            fny=pltpu.VMEM((fpt,), jnp.float32),
            fnz=pltpu.VMEM((fpt,), jnp.float32),
            vtx0=pltpu.VMEM((NV,), jnp.float32),
            vty0=pltpu.VMEM((NV,), jnp.float32),
            vtz0=pltpu.VMEM((NV,), jnp.float32),
            vtx1=pltpu.VMEM((NV,), jnp.float32),
            vty1=pltpu.VMEM((NV,), jnp.float32),
            vtz1=pltpu.VMEM((NV,), jnp.float32),
            zbuf=pltpu.VMEM((NV // NS,), jnp.float32),
            shx=pltpu.VMEM_SHARED((NV,), jnp.float32),
            shy=pltpu.VMEM_SHARED((NV,), jnp.float32),
            shz=pltpu.VMEM_SHARED((NV,), jnp.float32),
            sem=pltpu.SemaphoreType.DMA,
            sem2=pltpu.SemaphoreType.DMA,
        ),
    )
    def k(gv_hbm, gf_hbm, out_hbm, f0, f1, f2, fnx, fny, fnz,
          vtx0, vty0, vtz0, vtx1, vty1, vtz1, zbuf, shx, shy, shz,
          sem, sem2):
        core = lax.axis_index("c")
        sid = lax.axis_index("s")
        nsl = NV // NS
        vt = [(vtx0, vty0, vtz0), (vtx1, vty1, vtz1)]

        pltpu.sync_copy(gf_hbm.at[pl.ds(0 * FG + sid * fpt, fpt)], f0)
        pltpu.sync_copy(gf_hbm.at[pl.ds(1 * FG + sid * fpt, fpt)], f1)
        pltpu.sync_copy(gf_hbm.at[pl.ds(2 * FG + sid * fpt, fpt)], f2)

        for i in range(nsl // L):
            zbuf[pl.ds(i * L, L)] = jnp.zeros((L,), jnp.float32)

        def issue(lb):
            row = (core * 2 + lb) * 3
            return [pltpu.async_copy(gv_hbm.at[pl.ds((row + c) * NV, NV)],
                                     vt[lb][c], sem) for c in range(3)]

        descs = issue(0)
        for lb in range(2):
            b = core * 2 + lb
            row = b * 3
            pltpu.sync_copy(zbuf, shx.at[pl.ds(sid * nsl, nsl)])
            pltpu.sync_copy(zbuf, shy.at[pl.ds(sid * nsl, nsl)])
            pltpu.sync_copy(zbuf, shz.at[pl.ds(sid * nsl, nsl)])
            for d in descs:
                d.wait()
            if lb == 0:
                descs = issue(1)
            vtx, vty, vtz = vt[lb]
            plsc.subcore_barrier()

            def face_chunk(j, _):
                sl = pl.ds(j * L, L)
                a = f0[sl]
                bb = f1[sl]
                cc = f2[sl]
                v0x = plsc.load_gather(vtx, [a])
                v0y = plsc.load_gather(vty, [a])
                v0z = plsc.load_gather(vtz, [a])
                v1x = plsc.load_gather(vtx, [bb])
                v1y = plsc.load_gather(vty, [bb])
                v1z = plsc.load_gather(vtz, [bb])
                v2x = plsc.load_gather(vtx, [cc])
                v2y = plsc.load_gather(vty, [cc])
                v2z = plsc.load_gather(vtz, [cc])
                e1x, e1y, e1z = v1x - v0x, v1y - v0y, v1z - v0z
                e2x, e2y, e2z = v2x - v0x, v2y - v0y, v2z - v0z
                fnx[sl] = e1y * e2z - e1z * e2y
                fny[sl] = e1z * e2x - e1x * e2z
                fnz[sl] = e1x * e2y - e1y * e2x
                return 0

            lax.fori_loop(0, fpt // L, face_chunk, 0)

            adds = []
            for fidx in (f0, f1, f2):
                adds.append(pltpu.async_copy(fnx, shx.at[fidx], sem2,
                                             add=True))
                adds.append(pltpu.async_copy(fny, shy.at[fidx], sem2,
                                             add=True))
                adds.append(pltpu.async_copy(fnz, shz.at[fidx], sem2,
                                             add=True))
            for d in adds:
                d.wait()
            plsc.subcore_barrier()

            sl = pl.ds(sid * nsl, nsl)
            pltpu.sync_copy(shx.at[sl],
                            out_hbm.at[pl.ds((row + 0) * NV + sid * nsl, nsl)])
            pltpu.sync_copy(shy.at[sl],
                            out_hbm.at[pl.ds((row + 1) * NV + sid * nsl, nsl)])
            pltpu.sync_copy(shz.at[sl],
                            out_hbm.at[pl.ds((row + 2) * NV + sid * nsl, nsl)])
            plsc.subcore_barrier()

    return k(gv_flat, gf_flat)


BP = 1024
NVC = 2048


def _chamfer_tc(pv, gvt, nrm):
    def body(pv_ref, gvt_ref, nrm_ref, idx_ref, nhat_ref):
        p = pv_ref[0]
        g = gvt_ref[0]
        g2 = jnp.sum(g * g, axis=0, keepdims=True)
        p2 = jnp.sum(p * p, axis=1, keepdims=True)
        bf = jnp.bfloat16
        f32 = jnp.float32
        rg = -2.0 * g
        rgh = rg.astype(bf)
        rgl = (rg - rgh.astype(f32)).astype(bf)
        g2h = g2.astype(bf)
        g2l = (g2 - g2h.astype(f32)).astype(bf)
        ph = p.astype(bf)
        pl_ = (p - ph.astype(f32)).astype(bf)
        p2h = p2.astype(bf)
        p2l = (p2 - p2h.astype(f32)).astype(bf)
        ones_c = jnp.ones((BP, 1), bf)
        zeros_c = jnp.zeros((BP, 3), bf)
        lhs16 = jnp.concatenate(
            [ph, ph, pl_, ones_c, ones_c, p2h, p2l, zeros_c], axis=1)
        ones_r = jnp.ones((1, NV), bf)
        zeros_r = jnp.zeros((3, NV), bf)
        rhs16 = jnp.concatenate(
            [rgh, rgl, rgh, g2h, g2l, ones_r, ones_r, zeros_r], axis=0)
        dn = (((1,), (0,)), ((), ()))
        ids = lax.broadcasted_iota(jnp.int32, (BP, NVC), 1)
        mks = []
        for h in range(NV // NVC):
            d = lax.dot_general(lhs16, rhs16[:, h * NVC:(h + 1) * NVC], dn,
                                preferred_element_type=jnp.float32)
            bits = lax.bitcast_convert_type(d, jnp.int32)
            key = lax.bitcast_convert_type((bits & ~(NVC - 1)) | ids, jnp.float32)
            mks.append(jnp.min(key, axis=1, keepdims=True))
        m = mks[0]
        for mk in mks[1:]:
            m = jnp.minimum(m, mk)
        amin = None
        for h, mk in enumerate(mks):
            lid = lax.bitcast_convert_type(mk, jnp.int32) & (NVC - 1)
            cand = jnp.where(mk <= m, lid + h * NVC, NV)
            amin = cand if amin is None else jnp.minimum(amin, cand)
        idx_ref[0, 0] = jnp.transpose(amin, (1, 0))

        n = nrm_ref[0]
        ns = jnp.sqrt(jnp.sum(n * n, axis=0, keepdims=True))
        nhat_ref[0] = n / jnp.maximum(ns, 1e-12)

    return pl.pallas_call(
        body,
        grid=(B, NV // BP),
        in_specs=[
            pl.BlockSpec((1, BP, 3), lambda b, i: (b, i, 0)),
            pl.BlockSpec((1, 3, NV), lambda b, i: (b, 0, 0)),
            pl.BlockSpec((1, 3, BP), lambda b, i: (b, 0, i)),
        ],
        out_specs=[
            pl.BlockSpec((1, 1, 1, BP), lambda b, i: (b, i, 0, 0)),
            pl.BlockSpec((1, 3, BP), lambda b, i: (b, 0, i)),
        ],
        out_shape=[
            jax.ShapeDtypeStruct((B, NV // BP, 1, BP), jnp.int32),
            jax.ShapeDtypeStruct((B, 3, NV), jnp.float32),
        ],
    )(pv, gvt, nrm)


def _edge_loss_sc(ea, eb, ew, idx, pvt_flat, nhat_flat):
    epw = EU // NW
    psl = 3 * NV // NS
    isl = NV // NS
    mesh = plsc.VectorSubcoreMesh(core_axis_name="c", subcore_axis_name="s")

    @functools.partial(
        pl.kernel,
        out_type=jax.ShapeDtypeStruct((NW * L,), jnp.float32),
        mesh=mesh,
        compiler_params=pltpu.CompilerParams(needs_layout_passes=False),
        scratch_types=dict(
            eas=pltpu.VMEM((epw,), jnp.int32),
            ebs=pltpu.VMEM((epw,), jnp.int32),
            ews=pltpu.VMEM((epw,), jnp.float32),
            idxt=pltpu.VMEM((NV,), jnp.int32),
            px=pltpu.VMEM((NV,), jnp.float32),
            py=pltpu.VMEM((NV,), jnp.float32),
            pz=pltpu.VMEM((NV,), jnp.float32),
            nx=pltpu.VMEM((NV,), jnp.float32),
            ny=pltpu.VMEM((NV,), jnp.float32),
            nz=pltpu.VMEM((NV,), jnp.float32),
            accb=pltpu.VMEM((L,), jnp.float32),
            shp0=pltpu.VMEM_SHARED((3 * NV,), jnp.float32),
            shp1=pltpu.VMEM_SHARED((3 * NV,), jnp.float32),
            shn0=pltpu.VMEM_SHARED((3 * NV,), jnp.float32),
            shn1=pltpu.VMEM_SHARED((3 * NV,), jnp.float32),
            shi0=pltpu.VMEM_SHARED((NV,), jnp.int32),
            shi1=pltpu.VMEM_SHARED((NV,), jnp.int32),
            sem=pltpu.SemaphoreType.DMA,
        ),
    )
    def k(ea_hbm, eb_hbm, ew_hbm, idx_hbm, pv_hbm, nh_hbm, out_hbm,
          eas, ebs, ews, idxt, px, py, pz, nx, ny, nz, accb,
          shp0, shp1, shn0, shn1, shi0, shi1, sem):
        core = lax.axis_index("c")
        sid = lax.axis_index("s")
        wid = sid * NC + core
        base = wid * epw
        edescs = [
            pltpu.async_copy(ea_hbm.at[pl.ds(base, epw)], eas, sem),
            pltpu.async_copy(eb_hbm.at[pl.ds(base, epw)], ebs, sem),
            pltpu.async_copy(ew_hbm.at[pl.ds(base, epw)], ews, sem),
        ]
        shp = (shp0, shp1)
        shn = (shn0, shn1)
        shi = (shi0, shi1)

        def stripe(b):
            p = b % 2
            return [
                pltpu.async_copy(
                    pv_hbm.at[pl.ds(b * 3 * NV + sid * psl, psl)],
                    shp[p].at[pl.ds(sid * psl, psl)], sem),
                pltpu.async_copy(
                    nh_hbm.at[pl.ds(b * 3 * NV + sid * psl, psl)],
                    shn[p].at[pl.ds(sid * psl, psl)], sem),
                pltpu.async_copy(
                    idx_hbm.at[pl.ds(b * NV + sid * isl, isl)],
                    shi[p].at[pl.ds(sid * isl, isl)], sem),
            ]

        descs = stripe(0)
        for d in edescs:
            d.wait()

        acc = jnp.zeros((L,), jnp.float32)
        for b in range(B):
            p = b % 2
            for d in descs:
                d.wait()
            plsc.subcore_barrier()
            pltpu.sync_copy(shp[p].at[pl.ds(0 * NV, NV)], px)
            pltpu.sync_copy(shp[p].at[pl.ds(1 * NV, NV)], py)
            pltpu.sync_copy(shp[p].at[pl.ds(2 * NV, NV)], pz)
            pltpu.sync_copy(shn[p].at[pl.ds(0 * NV, NV)], nx)
            pltpu.sync_copy(shn[p].at[pl.ds(1 * NV, NV)], ny)
            pltpu.sync_copy(shn[p].at[pl.ds(2 * NV, NV)], nz)
            pltpu.sync_copy(shi[p], idxt)
            if b + 1 < B:
                descs = stripe(b + 1)

            def edge_chunk(j, acc):
                sl = pl.ds(j * L, L)
                a = eas[sl]
                bb = ebs[sl]
                w = ews[sl]
                ia = plsc.load_gather(idxt, [a])
                ib = plsc.load_gather(idxt, [bb])
                dx = plsc.load_gather(px, [a]) - plsc.load_gather(px, [bb])
                dy = plsc.load_gather(py, [a]) - plsc.load_gather(py, [bb])
                dz = plsc.load_gather(pz, [a]) - plsc.load_gather(pz, [bb])
                d1 = dx * plsc.load_gather(nx, [ia]) \
                    + dy * plsc.load_gather(ny, [ia]) \
                    + dz * plsc.load_gather(nz, [ia])
                d2 = dx * plsc.load_gather(nx, [ib]) \
                    + dy * plsc.load_gather(ny, [ib]) \
                    + dz * plsc.load_gather(nz, [ib])
                return acc + (jnp.abs(d1) + jnp.abs(d2)) * w

            acc = lax.fori_loop(0, epw // L, edge_chunk, acc)

        accb[...] = acc
        pltpu.sync_copy(accb, out_hbm.at[pl.ds(wid * L, L)])

    return k(ea, eb, ew, idx, pvt_flat, nhat_flat)


def kernel(pred_vertices, pred_faces, gt_vertices, gt_faces):
    pred_vertices = pred_vertices.astype(jnp.float32)
    gt_vertices = gt_vertices.astype(jnp.float32)

    ea, eb, ew, nuniq = _edge_prep(pred_faces)

    gv_flat = jnp.transpose(gt_vertices, (0, 2, 1)).reshape(-1)
    pvt = jnp.transpose(pred_vertices, (0, 2, 1))
    pvt_flat = pvt.reshape(-1)
    gf_flat = jnp.transpose(gt_faces.astype(jnp.int32), (1, 0)).reshape(-1)

    nrm_flat = _normals_sc(gv_flat, gf_flat)
    gvt = jnp.transpose(gt_vertices, (0, 2, 1))
    idx3, nhat = _chamfer_tc(pred_vertices, gvt, nrm_flat.reshape(B, 3, NV))
    partials = _edge_loss_sc(ea, eb, ew, idx3.reshape(-1),
                             pvt_flat, nhat.reshape(-1))

    denom = (B * 2 * nuniq).astype(jnp.float32)
    return jnp.sum(partials) / denom


# speedup vs baseline: 1.3228x; 1.0246x over previous
import functools

import jax
import jax.numpy as jnp
from jax import lax
from jax.experimental import pallas as pl
from jax.experimental.pallas import tpu as pltpu
from jax.experimental.pallas import tpu_sc as plsc

NC = 2
NS = 16
NW = NC * NS
L = 16

B = 4
NV = 4096
FG = 8192
FP = 8192
EU = 3 * FP


def _edge_prep(pred_faces):
    f = pred_faces.astype(jnp.int32)
    u = jnp.concatenate([f[:, 0], f[:, 1], f[:, 2]], axis=0)
    v = jnp.concatenate([f[:, 1], f[:, 2], f[:, 0]], axis=0)
    key = jnp.minimum(u, v) * NV + jnp.maximum(u, v)
    skey = jnp.sort(key)
    first = jnp.concatenate(
        [jnp.ones((1,), dtype=bool), skey[1:] != skey[:-1]])
    ea = skey >> 12
    eb = skey & (NV - 1)
    return ea, eb, first.astype(jnp.float32), jnp.sum(first)


def _normals_sc(gv_flat, gf_flat):
    fpt = FG // NS
    mesh = plsc.VectorSubcoreMesh(core_axis_name="c", subcore_axis_name="s")

    @functools.partial(
        pl.kernel,
        out_type=jax.ShapeDtypeStruct((B * 3 * NV,), jnp.float32),
        mesh=mesh,
        compiler_params=pltpu.CompilerParams(needs_layout_passes=False),
        scratch_types=dict(
            f0=pltpu.VMEM((fpt,), jnp.int32),
            f1=pltpu.VMEM((fpt,), jnp.int32),
            f2=pltpu.VMEM((fpt,), jnp.int32),
            fnx=pltpu.VMEM((fpt,), jnp.float32),
            fny=pltpu.VMEM((fpt,), jnp.float32),
            fnz=pltpu.VMEM((fpt,), jnp.float32),
            vtx0=pltpu.VMEM((NV,), jnp.float32),
            vty0=pltpu.VMEM((NV,), jnp.float32),
            vtz0=pltpu.VMEM((NV,), jnp.float32),
            vtx1=pltpu.VMEM((NV,), jnp.float32),
            vty1=pltpu.VMEM((NV,), jnp.float32),
            vtz1=pltpu.VMEM((NV,), jnp.float32),
            zbuf=pltpu.VMEM((NV // NS,), jnp.float32),
            shx=pltpu.VMEM_SHARED((NV,), jnp.float32),
            shy=pltpu.VMEM_SHARED((NV,), jnp.float32),
            shz=pltpu.VMEM_SHARED((NV,), jnp.float32),
            sem=pltpu.SemaphoreType.DMA,
            sem2=pltpu.SemaphoreType.DMA,
        ),
    )
    def k(gv_hbm, gf_hbm, out_hbm, f0, f1, f2, fnx, fny, fnz,
          vtx0, vty0, vtz0, vtx1, vty1, vtz1, zbuf, shx, shy, shz,
          sem, sem2):
        core = lax.axis_index("c")
        sid = lax.axis_index("s")
        nsl = NV // NS
        vt = [(vtx0, vty0, vtz0), (vtx1, vty1, vtz1)]

        pltpu.sync_copy(gf_hbm.at[pl.ds(0 * FG + sid * fpt, fpt)], f0)
        pltpu.sync_copy(gf_hbm.at[pl.ds(1 * FG + sid * fpt, fpt)], f1)
        pltpu.sync_copy(gf_hbm.at[pl.ds(2 * FG + sid * fpt, fpt)], f2)

        for i in range(nsl // L):
            zbuf[pl.ds(i * L, L)] = jnp.zeros((L,), jnp.float32)

        def issue(lb):
            row = (core * 2 + lb) * 3
            return [pltpu.async_copy(gv_hbm.at[pl.ds((row + c) * NV, NV)],
                                     vt[lb][c], sem) for c in range(3)]

        descs = issue(0)
        for lb in range(2):
            b = core * 2 + lb
            row = b * 3
            pltpu.sync_copy(zbuf, shx.at[pl.ds(sid * nsl, nsl)])
            pltpu.sync_copy(zbuf, shy.at[pl.ds(sid * nsl, nsl)])
            pltpu.sync_copy(zbuf, shz.at[pl.ds(sid * nsl, nsl)])
            for d in descs:
                d.wait()
            if lb == 0:
                descs = issue(1)
            vtx, vty, vtz = vt[lb]
            plsc.subcore_barrier()

            def face_chunk(j, _):
                sl = pl.ds(j * L, L)
                a = f0[sl]
                bb = f1[sl]
                cc = f2[sl]
                v0x = plsc.load_gather(vtx, [a])
                v0y = plsc.load_gather(vty, [a])
                v0z = plsc.load_gather(vtz, [a])
                v1x = plsc.load_gather(vtx, [bb])
                v1y = plsc.load_gather(vty, [bb])
                v1z = plsc.load_gather(vtz, [bb])
                v2x = plsc.load_gather(vtx, [cc])
                v2y = plsc.load_gather(vty, [cc])
                v2z = plsc.load_gather(vtz, [cc])
                e1x, e1y, e1z = v1x - v0x, v1y - v0y, v1z - v0z
                e2x, e2y, e2z = v2x - v0x, v2y - v0y, v2z - v0z
                fnx[sl] = e1y * e2z - e1z * e2y
                fny[sl] = e1z * e2x - e1x * e2z
                fnz[sl] = e1x * e2y - e1y * e2x
                return 0

            lax.fori_loop(0, fpt // L, face_chunk, 0)

            adds = []
            for fidx in (f0, f1, f2):
                adds.append(pltpu.async_copy(fnx, shx.at[fidx], sem2,
                                             add=True))
                adds.append(pltpu.async_copy(fny, shy.at[fidx], sem2,
                                             add=True))
                adds.append(pltpu.async_copy(fnz, shz.at[fidx], sem2,
                                             add=True))
            for d in adds:
                d.wait()
            plsc.subcore_barrier()

            sl = pl.ds(sid * nsl, nsl)
            pltpu.sync_copy(shx.at[sl], fnx.at[pl.ds(0, nsl)])
            pltpu.sync_copy(shy.at[sl], fny.at[pl.ds(0, nsl)])
            pltpu.sync_copy(shz.at[sl], fnz.at[pl.ds(0, nsl)])

            def norm_chunk(j, _):
                cs = pl.ds(j * L, L)
                x = fnx[cs]
                y = fny[cs]
                z = fnz[cs]
                nn = x * x + y * y + z * z
                bits = plsc.bitcast(nn, jnp.int32)
                r = plsc.bitcast(0x5F3759DF - (bits >> 1), jnp.float32)
                for _i in range(3):
                    r = r * (1.5 - 0.5 * nn * r * r)
                r = jnp.where(nn > 1e-35, r, 0.0)
                fnx[cs] = x * r
                fny[cs] = y * r
                fnz[cs] = z * r
                return 0

            lax.fori_loop(0, nsl // L, norm_chunk, 0)
            pltpu.sync_copy(fnx.at[pl.ds(0, nsl)],
                            out_hbm.at[pl.ds((row + 0) * NV + sid * nsl, nsl)])
            pltpu.sync_copy(fny.at[pl.ds(0, nsl)],
                            out_hbm.at[pl.ds((row + 1) * NV + sid * nsl, nsl)])
            pltpu.sync_copy(fnz.at[pl.ds(0, nsl)],
                            out_hbm.at[pl.ds((row + 2) * NV + sid * nsl, nsl)])
            plsc.subcore_barrier()

    return k(gv_flat, gf_flat)


BP = 1024
NVC = 2048


def _chamfer_tc(pv, gvt):
    def body(pv_ref, gvt_ref, idx_ref):
        p = pv_ref[0]
        g = gvt_ref[0]
        g2 = jnp.sum(g * g, axis=0, keepdims=True)
        p2 = jnp.sum(p * p, axis=1, keepdims=True)
        bf = jnp.bfloat16
        f32 = jnp.float32
        rg = -2.0 * g
        rgh = rg.astype(bf)
        rgl = (rg - rgh.astype(f32)).astype(bf)
        g2h = g2.astype(bf)
        g2l = (g2 - g2h.astype(f32)).astype(bf)
        ph = p.astype(bf)
        pl_ = (p - ph.astype(f32)).astype(bf)
        p2h = p2.astype(bf)
        p2l = (p2 - p2h.astype(f32)).astype(bf)
        ones_c = jnp.ones((BP, 1), bf)
        zeros_c = jnp.zeros((BP, 3), bf)
        lhs16 = jnp.concatenate(
            [ph, ph, pl_, ones_c, ones_c, p2h, p2l, zeros_c], axis=1)
        ones_r = jnp.ones((1, NV), bf)
        zeros_r = jnp.zeros((3, NV), bf)
        rhs16 = jnp.concatenate(
            [rgh, rgl, rgh, g2h, g2l, ones_r, ones_r, zeros_r], axis=0)
        dn = (((1,), (0,)), ((), ()))
        ids = lax.broadcasted_iota(jnp.int32, (BP, NVC), 1)
        mks = []
        for h in range(NV // NVC):
            d = lax.dot_general(lhs16, rhs16[:, h * NVC:(h + 1) * NVC], dn,
                                preferred_element_type=jnp.float32)
            bits = lax.bitcast_convert_type(d, jnp.int32)
            key = lax.bitcast_convert_type((bits & ~(NVC - 1)) | ids, jnp.float32)
            mks.append(jnp.min(key, axis=1, keepdims=True))
        m = mks[0]
        for mk in mks[1:]:
            m = jnp.minimum(m, mk)
        amin = None
        for h, mk in enumerate(mks):
            lid = lax.bitcast_convert_type(mk, jnp.int32) & (NVC - 1)
            cand = jnp.where(mk <= m, lid + h * NVC, NV)
            amin = cand if amin is None else jnp.minimum(amin, cand)
        idx_ref[0, 0] = jnp.transpose(amin, (1, 0))

    return pl.pallas_call(
        body,
        grid=(B, NV // BP),
        in_specs=[
            pl.BlockSpec((1, BP, 3), lambda b, i: (b, i, 0)),
            pl.BlockSpec((1, 3, NV), lambda b, i: (b, 0, 0)),
        ],
        out_specs=pl.BlockSpec((1, 1, 1, BP), lambda b, i: (b, i, 0, 0)),
        out_shape=jax.ShapeDtypeStruct((B, NV // BP, 1, BP), jnp.int32),
    )(pv, gvt)


def _edge_loss_sc(ea, eb, ew, idx, pvt_flat, nhat_flat):
    epw = EU // NW
    psl = 3 * NV // NS
    isl = NV // NS
    mesh = plsc.VectorSubcoreMesh(core_axis_name="c", subcore_axis_name="s")

    @functools.partial(
        pl.kernel,
        out_type=jax.ShapeDtypeStruct((NW * L,), jnp.float32),
        mesh=mesh,
        compiler_params=pltpu.CompilerParams(needs_layout_passes=False),
        scratch_types=dict(
            eas=pltpu.VMEM((epw,), jnp.int32),
            ebs=pltpu.VMEM((epw,), jnp.int32),
            ews=pltpu.VMEM((epw,), jnp.float32),
            idxt=pltpu.VMEM((NV,), jnp.int32),
            px=pltpu.VMEM((NV,), jnp.float32),
            py=pltpu.VMEM((NV,), jnp.float32),
            pz=pltpu.VMEM((NV,), jnp.float32),
            nx=pltpu.VMEM((NV,), jnp.float32),
            ny=pltpu.VMEM((NV,), jnp.float32),
            nz=pltpu.VMEM((NV,), jnp.float32),
            accb=pltpu.VMEM((L,), jnp.float32),
            shp0=pltpu.VMEM_SHARED((3 * NV,), jnp.float32),
            shp1=pltpu.VMEM_SHARED((3 * NV,), jnp.float32),
            shn0=pltpu.VMEM_SHARED((3 * NV,), jnp.float32),
            shn1=pltpu.VMEM_SHARED((3 * NV,), jnp.float32),
            shi0=pltpu.VMEM_SHARED((NV,), jnp.int32),
            shi1=pltpu.VMEM_SHARED((NV,), jnp.int32),
            sem=pltpu.SemaphoreType.DMA,
        ),
    )
    def k(ea_hbm, eb_hbm, ew_hbm, idx_hbm, pv_hbm, nh_hbm, out_hbm,
          eas, ebs, ews, idxt, px, py, pz, nx, ny, nz, accb,
          shp0, shp1, shn0, shn1, shi0, shi1, sem):
        core = lax.axis_index("c")
        sid = lax.axis_index("s")
        wid = sid * NC + core
        base = wid * epw
        edescs = [
            pltpu.async_copy(ea_hbm.at[pl.ds(base, epw)], eas, sem),
            pltpu.async_copy(eb_hbm.at[pl.ds(base, epw)], ebs, sem),
            pltpu.async_copy(ew_hbm.at[pl.ds(base, epw)], ews, sem),
        ]
        shp = (shp0, shp1)
        shn = (shn0, shn1)
        shi = (shi0, shi1)

        def stripe(b):
            p = b % 2
            return [
                pltpu.async_copy(
                    pv_hbm.at[pl.ds(b * 3 * NV + sid * psl, psl)],
                    shp[p].at[pl.ds(sid * psl, psl)], sem),
                pltpu.async_copy(
                    nh_hbm.at[pl.ds(b * 3 * NV + sid * psl, psl)],
                    shn[p].at[pl.ds(sid * psl, psl)], sem),
                pltpu.async_copy(
                    idx_hbm.at[pl.ds(b * NV + sid * isl, isl)],
                    shi[p].at[pl.ds(sid * isl, isl)], sem),
            ]

        descs = stripe(0)
        for d in edescs:
            d.wait()

        acc = jnp.zeros((L,), jnp.float32)
        for b in range(B):
            p = b % 2
            for d in descs:
                d.wait()
            plsc.subcore_barrier()
            pltpu.sync_copy(shp[p].at[pl.ds(0 * NV, NV)], px)
            pltpu.sync_copy(shp[p].at[pl.ds(1 * NV, NV)], py)
            pltpu.sync_copy(shp[p].at[pl.ds(2 * NV, NV)], pz)
            pltpu.sync_copy(shn[p].at[pl.ds(0 * NV, NV)], nx)
            pltpu.sync_copy(shn[p].at[pl.ds(1 * NV, NV)], ny)
            pltpu.sync_copy(shn[p].at[pl.ds(2 * NV, NV)], nz)
            pltpu.sync_copy(shi[p], idxt)
            if b + 1 < B:
                descs = stripe(b + 1)

            def edge_chunk(j, acc):
                sl = pl.ds(j * L, L)
                a = eas[sl]
                bb = ebs[sl]
                w = ews[sl]
                ia = plsc.load_gather(idxt, [a])
                ib = plsc.load_gather(idxt, [bb])
                dx = plsc.load_gather(px, [a]) - plsc.load_gather(px, [bb])
                dy = plsc.load_gather(py, [a]) - plsc.load_gather(py, [bb])
                dz = plsc.load_gather(pz, [a]) - plsc.load_gather(pz, [bb])
                d1 = dx * plsc.load_gather(nx, [ia]) \
                    + dy * plsc.load_gather(ny, [ia]) \
                    + dz * plsc.load_gather(nz, [ia])
                d2 = dx * plsc.load_gather(nx, [ib]) \
                    + dy * plsc.load_gather(ny, [ib]) \
                    + dz * plsc.load_gather(nz, [ib])
                return acc + (jnp.abs(d1) + jnp.abs(d2)) * w

            acc = lax.fori_loop(0, epw // L, edge_chunk, acc)

        accb[...] = acc
        pltpu.sync_copy(accb, out_hbm.at[pl.ds(wid * L, L)])

    return k(ea, eb, ew, idx, pvt_flat, nhat_flat)


def kernel(pred_vertices, pred_faces, gt_vertices, gt_faces):
    pred_vertices = pred_vertices.astype(jnp.float32)
    gt_vertices = gt_vertices.astype(jnp.float32)

    ea, eb, ew, nuniq = _edge_prep(pred_faces)

    gv_flat = jnp.transpose(gt_vertices, (0, 2, 1)).reshape(-1)
    pvt = jnp.transpose(pred_vertices, (0, 2, 1))
    pvt_flat = pvt.reshape(-1)
    gf_flat = jnp.transpose(gt_faces.astype(jnp.int32), (1, 0)).reshape(-1)

    nhat_flat = _normals_sc(gv_flat, gf_flat)
    gvt = jnp.transpose(gt_vertices, (0, 2, 1))
    idx3 = _chamfer_tc(pred_vertices, gvt)
    partials = _edge_loss_sc(ea, eb, ew, idx3.reshape(-1),
                             pvt_flat, nhat_flat)

    denom = (B * 2 * nuniq).astype(jnp.float32)
    return jnp.sum(partials) / denom
